# baseline (speedup 1.0000x reference)
import functools

import jax
import jax.numpy as jnp
from jax import lax
from jax.experimental import pallas as pl
from jax.experimental.pallas import tpu as pltpu
from jax.experimental.pallas import tpu_sc as plsc

_NC = 2
_NS = 16
_NW = _NC * _NS
_GC = 256
_BC = 128


def kernel(data, indices):
    vocab, dim = data.shape
    batch, hist = indices.shape
    data_t = data.T
    idx_t = indices.T

    bw = batch // _NW
    nsub = bw // _BC
    vfull = (vocab // _GC) * _GC
    ngrp = vfull // _GC
    gpw = -(-ngrp // _NS)
    vtail = vocab - vfull
    tail_lin = data[vfull:, :].reshape(vtail * dim)

    mesh = plsc.VectorSubcoreMesh(core_axis_name="c", subcore_axis_name="s")

    @functools.partial(
        pl.kernel,
        mesh=mesh,
        compiler_params=pltpu.CompilerParams(
            use_tc_tiling_on_sc=True, needs_layout_passes=False),
        out_type=jax.ShapeDtypeStruct((hist, dim, batch), jnp.float32),
        scratch_types=[
            pltpu.HBM((vocab, dim), jnp.float32),
            pltpu.VMEM((dim, _GC), jnp.float32),
            pltpu.VMEM((_GC, dim), jnp.float32),
            pltpu.VMEM((vtail * dim,), jnp.float32),
            pltpu.VMEM((hist, bw), jnp.int32),
            pltpu.VMEM((_BC,), jnp.int32),
            pltpu.VMEM((_BC, dim), jnp.float32),
            pltpu.VMEM((dim, _BC), jnp.float32),
            pltpu.SemaphoreType.DMA,
        ],
    )
    def gather_k(tab_hbm, idx_hbm, tail_hbm, out_hbm, scr_hbm,
                 blk_v, tblk_v, tail_v, idx_v, idxr_v, rows_v, slab_v, sem):
        c = lax.axis_index("c")
        s = lax.axis_index("s")
        wid = s * _NC + c
        lane = lax.iota(jnp.int32, 16)

        d_lo = lane
        d_hi = lane + 16

        @pl.loop(0, gpw)
        def _grp(i):
            g = s + _NS * i

            @pl.when(g < ngrp)
            def _():
                v0 = pl.multiple_of(g * _GC, _GC)
                for k in range(4):
                    pltpu.sync_copy(
                        tab_hbm.at[pl.ds(8 * k, 8), pl.ds(v0, _GC)],
                        blk_v.at[pl.ds(8 * k, 8), :])

                @pl.loop(0, _GC)
                def _row(j):
                    jv = jnp.full((16,), j, jnp.int32)
                    tblk_v[j, pl.ds(0, 16)] = plsc.load_gather(
                        blk_v, [d_lo, jv])
                    tblk_v[j, pl.ds(16, 16)] = plsc.load_gather(
                        blk_v, [d_hi, jv])

                pltpu.sync_copy(tblk_v, scr_hbm.at[pl.ds(v0, _GC), :])

        if vtail:
            @pl.when(s == _NS - 1)
            def _tail():
                pltpu.sync_copy(tail_hbm, tail_v)
                for r in range(vtail):
                    pltpu.sync_copy(tail_v.at[pl.ds(r * dim, dim)],
                                    scr_hbm.at[vfull + r, :])

        plsc.subcore_barrier()

        b0 = pl.multiple_of(wid * bw, bw)
        pltpu.sync_copy(idx_hbm.at[:, pl.ds(b0, bw)], idx_v)

        @pl.loop(0, hist)
        def _slot(l):
            for m in range(nsub):
                @pl.loop(0, _BC // 16)
                def _vec(j):
                    idxr_v[pl.ds(j * 16, 16)] = idx_v[
                        l, pl.ds(m * _BC + j * 16, 16)]

                pltpu.async_copy(scr_hbm.at[idxr_v], rows_v, sem).wait()

                @pl.loop(0, dim)
                def _d(d):
                    dv = jnp.full((16,), d, jnp.int32)

                    @pl.loop(0, _BC // 16)
                    def _j(j2):
                        jv = lane + j2 * 16
                        slab_v[d, pl.ds(j2 * 16, 16)] = plsc.load_gather(
                            rows_v, [jv, dv])

                pltpu.sync_copy(
                    slab_v, out_hbm.at[l, :, pl.ds(b0 + m * _BC, _BC)])

    out_t = gather_k(data_t, idx_t, tail_lin)
    return jnp.transpose(out_t, (2, 0, 1))


# speedup vs baseline: 1.5460x; 1.5460x over previous
import functools

import jax
import jax.numpy as jnp
from jax import lax
from jax.experimental import pallas as pl
from jax.experimental.pallas import tpu as pltpu
from jax.experimental.pallas import tpu_sc as plsc

_NC = 2
_NS = 16
_NW = _NC * _NS
_GC = 128
_BC = 256


def kernel(data, indices):
    vocab, dim = data.shape
    batch, hist = indices.shape
    data_t = data.T
    idx_t = indices.T

    bw = batch // _NW
    nsub = bw // _BC
    nck = hist * nsub
    ngrp = vocab // _GC
    gmain = (ngrp // _NS) * _NS
    gpt = gmain // _NS
    vtail = vocab - ngrp * _GC
    tail_lin = data[ngrp * _GC:, :].reshape(vtail * dim)

    mesh = plsc.VectorSubcoreMesh(core_axis_name="c", subcore_axis_name="s")

    @functools.partial(
        pl.kernel,
        mesh=mesh,
        compiler_params=pltpu.CompilerParams(
            use_tc_tiling_on_sc=True, needs_layout_passes=False),
        out_type=jax.ShapeDtypeStruct((hist, dim, batch), jnp.float32),
        scratch_types=[
            pltpu.HBM((vocab, dim), jnp.float32),
            pltpu.VMEM((dim, _GC), jnp.float32),
            pltpu.VMEM((dim, _GC), jnp.float32),
            pltpu.VMEM((_GC, dim), jnp.float32),
            pltpu.VMEM((_GC, dim), jnp.float32),
            pltpu.VMEM((vtail * dim,), jnp.float32),
            pltpu.VMEM((_BC,), jnp.int32),
            pltpu.VMEM((_BC,), jnp.int32),
            pltpu.VMEM((_BC, dim), jnp.float32),
            pltpu.VMEM((_BC, dim), jnp.float32),
            pltpu.VMEM((dim, _BC), jnp.float32),
            pltpu.VMEM((dim, _BC), jnp.float32),
            pltpu.SemaphoreType.DMA,
            pltpu.SemaphoreType.DMA,
            pltpu.SemaphoreType.DMA,
            pltpu.SemaphoreType.DMA,
            pltpu.SemaphoreType.DMA,
            pltpu.SemaphoreType.DMA,
            pltpu.SemaphoreType.DMA,
            pltpu.SemaphoreType.DMA,
        ],
    )
    def gather_k(tab_hbm, idx_hbm, tail_hbm, out_hbm, scr_hbm,
                 blk0, blk1, tbk0, tbk1, tail_v, ixr0, ixr1,
                 row0, row1, slb0, slb1,
                 sr0, sr1, sw0, sw1, si0, si1, sg0, sg1):
        c = lax.axis_index("c")
        s = lax.axis_index("s")
        wid = s * _NC + c
        lane = lax.iota(jnp.int32, 16)
        blk = (blk0, blk1)
        tbk = (tbk0, tbk1)
        ixr = (ixr0, ixr1)
        row = (row0, row1)
        slb = (slb0, slb1)
        sr = (sr0, sr1)
        sw = (sw0, sw1)
        si = (si0, si1)
        sg = (sg0, sg1)

        d_lo = lane
        d_hi = lane + 16

        def grp_of(i):
            return s + _NS * i

        def rd_start(i, b):
            v0 = pl.multiple_of(grp_of(i) * _GC, _GC)
            pltpu.async_copy(tab_hbm.at[:, pl.ds(v0, _GC)], blk[b], sr[b])

        def rd_wait(b):
            pltpu.make_async_copy(
                tab_hbm.at[:, pl.ds(0, _GC)], blk[b], sr[b]).wait()

        def wr_start(i, b):
            v0 = pl.multiple_of(grp_of(i) * _GC, _GC)
            pltpu.async_copy(tbk[b], scr_hbm.at[pl.ds(v0, _GC), :], sw[b])

        def wr_wait(b):
            pltpu.make_async_copy(
                tbk[b], scr_hbm.at[pl.ds(0, _GC), :], sw[b]).wait()

        def transpose(b):
            @pl.loop(0, _GC, unroll=8)
            def _row(j):
                jv = jnp.full((16,), j, jnp.int32)
                tbk[b][j, pl.ds(0, 16)] = plsc.load_gather(blk[b], [d_lo, jv])
                tbk[b][j, pl.ds(16, 16)] = plsc.load_gather(blk[b], [d_hi, jv])

        rd_start(0, 0)

        @pl.loop(0, gpt // 2)
        def _pipe(t):
            for b in range(2):
                i = 2 * t + b
                rd_wait(b)

                @pl.when(i + 1 < gpt)
                def _():
                    rd_start(i + 1, 1 - b)

                @pl.when(i >= 2)
                def _():
                    wr_wait(b)

                transpose(b)
                wr_start(i, b)

        wr_wait(0)
        wr_wait(1)

        @pl.when(s < ngrp - gmain)
        def _rem():
            v0 = pl.multiple_of((gmain + s) * _GC, _GC)
            pltpu.sync_copy(tab_hbm.at[:, pl.ds(v0, _GC)], blk0)

            @pl.loop(0, _GC, unroll=8)
            def _row(j):
                jv = jnp.full((16,), j, jnp.int32)
                tbk0[j, pl.ds(0, 16)] = plsc.load_gather(blk0, [d_lo, jv])
                tbk0[j, pl.ds(16, 16)] = plsc.load_gather(blk0, [d_hi, jv])

            pltpu.sync_copy(tbk0, scr_hbm.at[pl.ds(v0, _GC), :])

        if vtail:
            @pl.when(s == _NS - 1)
            def _tail():
                pltpu.sync_copy(tail_hbm, tail_v)
                for r in range(vtail):
                    pltpu.sync_copy(tail_v.at[pl.ds(r * dim, dim)],
                                    scr_hbm.at[ngrp * _GC + r, :])

        plsc.subcore_barrier()

        b0 = pl.multiple_of(wid * bw, bw)

        def ix_start(k, b):
            l = k // nsub
            boff = pl.multiple_of(b0 + (k % nsub) * _BC, _BC)
            pltpu.async_copy(idx_hbm.at[l, pl.ds(boff, _BC)], ixr[b], si[b])

        def ix_wait(b):
            pltpu.make_async_copy(
                idx_hbm.at[0, pl.ds(0, _BC)], ixr[b], si[b]).wait()

        def g_start(b):
            pltpu.async_copy(scr_hbm.at[ixr[b]], row[b], sg[b])

        def g_wait(b):
            pltpu.make_async_copy(
                scr_hbm.at[ixr[b]], row[b], sg[b]).wait()

        def s_start(k, b):
            l = k // nsub
            boff = pl.multiple_of(b0 + (k % nsub) * _BC, _BC)
            pltpu.async_copy(slb[b], out_hbm.at[l, :, pl.ds(boff, _BC)], sw[b])

        def s_wait(b):
            pltpu.make_async_copy(
                slb[b], out_hbm.at[0, :, pl.ds(0, _BC)], sw[b]).wait()

        def slab_transpose(b):
            @pl.loop(0, dim)
            def _d(d):
                dv = jnp.full((16,), d, jnp.int32)

                @pl.loop(0, _BC // 16, unroll=8)
                def _j(j2):
                    jv = lane + j2 * 16
                    slb[b][d, pl.ds(j2 * 16, 16)] = plsc.load_gather(
                        row[b], [jv, dv])

        ix_start(0, 0)
        ix_wait(0)
        g_start(0)
        ix_start(1, 1)

        @pl.loop(0, nck // 2)
        def _p2(t):
            for b in range(2):
                k = 2 * t + b
                g_wait(b)

                @pl.when(k + 1 < nck)
                def _():
                    ix_wait(1 - b)
                    g_start(1 - b)

                @pl.when(k + 2 < nck)
                def _():
                    ix_start(k + 2, b)

                @pl.when(k >= 2)
                def _():
                    s_wait(b)

                slab_transpose(b)
                s_start(k, b)

        s_wait(0)
        s_wait(1)

    out_t = gather_k(data_t, idx_t, tail_lin)
    return jnp.transpose(out_t, (2, 0, 1))


# speedup vs baseline: 4.0599x; 2.6261x over previous
import functools

import jax
import jax.numpy as jnp
from jax import lax
from jax.experimental import pallas as pl
from jax.experimental.pallas import tpu as pltpu
from jax.experimental.pallas import tpu_sc as plsc

_NC = 2
_NS = 16
_NW = _NC * _NS
_GC = 128
_BC = 256


def kernel(data, indices):
    vocab, dim = data.shape
    batch, hist = indices.shape
    data_t = data.T
    idx_t = indices.T

    bw = batch // _NW
    nsub = bw // _BC
    nck = hist * nsub
    ngrp = vocab // _GC
    gmain = (ngrp // _NS) * _NS
    gpt = gmain // _NS
    vtail = vocab - ngrp * _GC
    tail_lin = data[ngrp * _GC:, :].reshape(vtail * dim)

    mesh = plsc.VectorSubcoreMesh(core_axis_name="c", subcore_axis_name="s")

    @functools.partial(
        pl.kernel,
        mesh=mesh,
        compiler_params=pltpu.CompilerParams(
            use_tc_tiling_on_sc=True, needs_layout_passes=False),
        out_type=jax.ShapeDtypeStruct((hist, dim, batch), jnp.float32),
        scratch_types=[
            pltpu.HBM((vocab, dim), jnp.float32),
            pltpu.VMEM((dim, _GC), jnp.float32),
            pltpu.VMEM((dim, _GC), jnp.float32),
            pltpu.VMEM((_GC, dim), jnp.float32),
            pltpu.VMEM((_GC, dim), jnp.float32),
            pltpu.VMEM((vtail * dim,), jnp.float32),
            pltpu.VMEM((_BC,), jnp.int32),
            pltpu.VMEM((_BC,), jnp.int32),
            pltpu.VMEM((_BC, dim), jnp.float32),
            pltpu.VMEM((_BC, dim), jnp.float32),
            pltpu.VMEM((dim, _BC), jnp.float32),
            pltpu.VMEM((dim, _BC), jnp.float32),
            pltpu.SemaphoreType.DMA,
            pltpu.SemaphoreType.DMA,
            pltpu.SemaphoreType.DMA,
            pltpu.SemaphoreType.DMA,
            pltpu.SemaphoreType.DMA,
            pltpu.SemaphoreType.DMA,
            pltpu.SemaphoreType.DMA,
            pltpu.SemaphoreType.DMA,
        ],
    )
    def gather_k(tab_hbm, idx_hbm, tail_hbm, out_hbm, scr_hbm,
                 blk0, blk1, tbk0, tbk1, tail_v, ixr0, ixr1,
                 row0, row1, slb0, slb1,
                 sr0, sr1, sw0, sw1, si0, si1, sg0, sg1):
        c = lax.axis_index("c")
        s = lax.axis_index("s")
        wid = s * _NC + c
        lane = lax.iota(jnp.int32, 16)
        blk = (blk0, blk1)
        tbk = (tbk0, tbk1)
        ixr = (ixr0, ixr1)
        row = (row0, row1)
        slb = (slb0, slb1)
        sr = (sr0, sr1)
        sw = (sw0, sw1)
        si = (si0, si1)
        sg = (sg0, sg1)

        d_lo = lane
        d_hi = lane + 16

        def grp_of(i):
            return s + _NS * i

        def rd_start(i, b):
            v0 = pl.multiple_of(grp_of(i) * _GC, _GC)
            pltpu.async_copy(tab_hbm.at[:, pl.ds(v0, _GC)], blk[b], sr[b])

        def rd_wait(b):
            pltpu.make_async_copy(
                tab_hbm.at[:, pl.ds(0, _GC)], blk[b], sr[b]).wait()

        def wr_start(i, b):
            v0 = pl.multiple_of(grp_of(i) * _GC, _GC)
            pltpu.async_copy(tbk[b], scr_hbm.at[pl.ds(v0, _GC), :], sw[b])

        def wr_wait(b):
            pltpu.make_async_copy(
                tbk[b], scr_hbm.at[pl.ds(0, _GC), :], sw[b]).wait()

        def transpose(b):
            @pl.loop(0, _GC // 16)
            def _rb(rb):
                rbase = rb * 16
                for dv in (d_lo, d_hi):
                    @pl.loop(0, 16, unroll=8)
                    def _q(q):
                        jv = rbase + ((q + lane) & 15)
                        plsc.store_scatter(
                            tbk[b], [jv, dv],
                            plsc.load_gather(blk[b], [dv, jv]))

        rd_start(0, 0)

        @pl.loop(0, gpt // 2)
        def _pipe(t):
            for b in range(2):
                i = 2 * t + b
                rd_wait(b)

                @pl.when(i + 1 < gpt)
                def _():
                    rd_start(i + 1, 1 - b)

                @pl.when(i >= 2)
                def _():
                    wr_wait(b)

                transpose(b)
                wr_start(i, b)

        wr_wait(0)
        wr_wait(1)

        @pl.when(s < ngrp - gmain)
        def _rem():
            v0 = pl.multiple_of((gmain + s) * _GC, _GC)
            pltpu.sync_copy(tab_hbm.at[:, pl.ds(v0, _GC)], blk0)
            transpose(0)
            pltpu.sync_copy(tbk0, scr_hbm.at[pl.ds(v0, _GC), :])

        if vtail:
            @pl.when(s == _NS - 1)
            def _tail():
                pltpu.sync_copy(tail_hbm, tail_v)
                for r in range(vtail):
                    pltpu.sync_copy(tail_v.at[pl.ds(r * dim, dim)],
                                    scr_hbm.at[ngrp * _GC + r, :])

        plsc.subcore_barrier()

        b0 = pl.multiple_of(wid * bw, bw)

        def ix_start(k, b):
            l = k // nsub
            boff = pl.multiple_of(b0 + (k % nsub) * _BC, _BC)
            pltpu.async_copy(idx_hbm.at[l, pl.ds(boff, _BC)], ixr[b], si[b])

        def ix_wait(b):
            pltpu.make_async_copy(
                idx_hbm.at[0, pl.ds(0, _BC)], ixr[b], si[b]).wait()

        def g_start(b):
            pltpu.async_copy(scr_hbm.at[ixr[b]], row[b], sg[b])

        def g_wait(b):
            pltpu.make_async_copy(
                scr_hbm.at[ixr[b]], row[b], sg[b]).wait()

        def s_start(k, b):
            l = k // nsub
            boff = pl.multiple_of(b0 + (k % nsub) * _BC, _BC)
            pltpu.async_copy(slb[b], out_hbm.at[l, :, pl.ds(boff, _BC)], sw[b])

        def s_wait(b):
            pltpu.make_async_copy(
                slb[b], out_hbm.at[0, :, pl.ds(0, _BC)], sw[b]).wait()

        def slab_transpose(b):
            @pl.loop(0, _BC // 16)
            def _jb(jb):
                jbase = jb * 16
                for dv in (d_lo, d_hi):
                    @pl.loop(0, 16, unroll=8)
                    def _q(q):
                        jv = jbase + ((q + lane) & 15)
                        plsc.store_scatter(
                            slb[b], [dv, jv],
                            plsc.load_gather(row[b], [jv, dv]))

        ix_start(0, 0)
        ix_wait(0)
        g_start(0)
        ix_start(1, 1)

        @pl.loop(0, nck // 2)
        def _p2(t):
            for b in range(2):
                k = 2 * t + b
                g_wait(b)

                @pl.when(k + 1 < nck)
                def _():
                    ix_wait(1 - b)
                    g_start(1 - b)

                @pl.when(k + 2 < nck)
                def _():
                    ix_start(k + 2, b)

                @pl.when(k >= 2)
                def _():
                    s_wait(b)

                slab_transpose(b)
                s_start(k, b)

        s_wait(0)
        s_wait(1)

    out_t = gather_k(data_t, idx_t, tail_lin)
    return jnp.transpose(out_t, (2, 0, 1))
